# baseline (speedup 1.0000x reference)
import functools

import jax
import jax.numpy as jnp
from jax.experimental import pallas as pl
from jax.experimental.pallas import tpu as pltpu


def _warp_block(q_ref, p_ref, f_ref, o_ref, *, n1):
    q = q_ref[0]
    p = p_ref[0]
    f = f_ref[0]

    qp = jax.lax.dot_general(
        q.astype(jnp.bfloat16), p.astype(jnp.bfloat16),
        (((1,), (0,)), ((), ())),
        preferred_element_type=jnp.float32,
    )
    qp_hi = jax.lax.dot_general(
        q, p, (((1,), (0,)), ((), ())),
        preferred_element_type=jnp.float32,
        precision=jax.lax.Precision.HIGHEST,
    )
    p2 = jnp.sum(p * p, axis=0, keepdims=True)
    q2 = jnp.sum(q * q, axis=1, keepdims=True)
    d = (q2 + p2) - 2.0 * qp
    d_w = (q2 + p2) - 2.0 * qp_hi

    iota = jax.lax.broadcasted_iota(jnp.int32, d.shape, 1)
    work = d
    r = jnp.zeros_like(d)
    norm = jnp.zeros((q.shape[0], 1), jnp.float32)
    for _ in range(3):
        mk = jnp.min(work, axis=1, keepdims=True)
        ik = jnp.min(jnp.where(work == mk, iota, n1), axis=1, keepdims=True)
        m_eq = iota == ik
        ek = jnp.min(jnp.where(m_eq, d_w, jnp.inf), axis=1, keepdims=True)
        rk = jax.lax.rsqrt(jnp.maximum(ek, 1e-20))
        r = jnp.where(m_eq, rk, r)
        norm = norm + rk
        work = jnp.where(m_eq, jnp.inf, work)
    flow2 = (
        jax.lax.dot_general(
            r, f, (((1,), (0,)), ((), ())),
            preferred_element_type=jnp.float32,
            precision=jax.lax.Precision.HIGHEST,
        )
        / norm
    )
    o_ref[0] = q - flow2


@jax.jit
def kernel(xyz1, xyz2, flow1):
    B, C, N1 = xyz1.shape
    N2 = xyz2.shape[2]
    mq = 512

    p_nat = xyz1 + flow1
    q_t = xyz2.transpose(0, 2, 1)
    f_t = flow1.transpose(0, 2, 1)

    out = pl.pallas_call(
        functools.partial(_warp_block, n1=N1),
        grid=(B, N2 // mq),
        in_specs=[
            pl.BlockSpec((1, mq, C), lambda b, i: (b, i, 0)),
            pl.BlockSpec((1, C, N1), lambda b, i: (b, 0, 0)),
            pl.BlockSpec((1, N1, C), lambda b, i: (b, 0, 0)),
        ],
        out_specs=pl.BlockSpec((1, mq, C), lambda b, i: (b, i, 0)),
        out_shape=jax.ShapeDtypeStruct((B, N2, C), jnp.float32),
        compiler_params=pltpu.CompilerParams(
            dimension_semantics=("parallel", "parallel")
        ),
    )(q_t, p_nat, f_t)
    return out.transpose(0, 2, 1)


# speedup vs baseline: 1.6583x; 1.6583x over previous
import functools

import jax
import jax.numpy as jnp
from jax.experimental import pallas as pl
from jax.experimental.pallas import tpu as pltpu


def _warp_block(q_ref, p_ref, f_ref, o_ref, *, n1):
    q = q_ref[0]
    p = p_ref[0]
    f = f_ref[0]

    qp = jax.lax.dot_general(
        q.astype(jnp.bfloat16), p.astype(jnp.bfloat16),
        (((1,), (0,)), ((), ())),
        preferred_element_type=jnp.float32,
    )
    p2 = jnp.sum(p * p, axis=0, keepdims=True)
    q2 = jnp.sum(q * q, axis=1, keepdims=True)
    d = (q2 + p2) - 2.0 * qp
    dx = q[:, 0:1] - p[0:1, :]
    dy = q[:, 1:2] - p[1:2, :]
    dz = q[:, 2:3] - p[2:3, :]
    d_w = (dx * dx + dy * dy) + dz * dz

    iota = jax.lax.broadcasted_iota(jnp.int32, d.shape, 1)
    work = d
    r = jnp.zeros_like(d)
    norm = jnp.zeros((q.shape[0], 1), jnp.float32)
    for _ in range(3):
        ik = jnp.argmin(work, axis=1)[:, None]
        m_eq = iota == ik
        ek = jnp.min(jnp.where(m_eq, d_w, jnp.inf), axis=1, keepdims=True)
        rk = jax.lax.rsqrt(jnp.maximum(ek, 1e-20))
        r = jnp.where(m_eq, rk, r)
        norm = norm + rk
        work = jnp.where(m_eq, jnp.inf, work)
    flow2 = (
        jax.lax.dot_general(
            r, f, (((1,), (0,)), ((), ())),
            preferred_element_type=jnp.float32,
            precision=jax.lax.Precision.HIGHEST,
        )
        / norm
    )
    o_ref[0] = q - flow2


@jax.jit
def kernel(xyz1, xyz2, flow1):
    B, C, N1 = xyz1.shape
    N2 = xyz2.shape[2]
    mq = 512

    p_nat = xyz1 + flow1
    q_t = xyz2.transpose(0, 2, 1)
    f_t = flow1.transpose(0, 2, 1)

    out = pl.pallas_call(
        functools.partial(_warp_block, n1=N1),
        grid=(B, N2 // mq),
        in_specs=[
            pl.BlockSpec((1, mq, C), lambda b, i: (b, i, 0)),
            pl.BlockSpec((1, C, N1), lambda b, i: (b, 0, 0)),
            pl.BlockSpec((1, N1, C), lambda b, i: (b, 0, 0)),
        ],
        out_specs=pl.BlockSpec((1, mq, C), lambda b, i: (b, i, 0)),
        out_shape=jax.ShapeDtypeStruct((B, N2, C), jnp.float32),
        compiler_params=pltpu.CompilerParams(
            dimension_semantics=("parallel", "parallel")
        ),
    )(q_t, p_nat, f_t)
    return out.transpose(0, 2, 1)


# speedup vs baseline: 2.7615x; 1.6652x over previous
import functools

import jax
import jax.numpy as jnp
from jax.experimental import pallas as pl
from jax.experimental.pallas import tpu as pltpu


def _warp_block(q_ref, p_ref, f_ref, o_ref, *, n1):
    q = q_ref[0]
    p = p_ref[0]
    f = f_ref[0]

    qp = jax.lax.dot_general(
        q.astype(jnp.bfloat16), p.astype(jnp.bfloat16),
        (((1,), (0,)), ((), ())),
        preferred_element_type=jnp.float32,
    )
    p2 = jnp.sum(p * p, axis=0, keepdims=True)
    q2 = jnp.sum(q * q, axis=1, keepdims=True)
    d = (q2 + p2) - 2.0 * qp
    dx = q[:, 0:1] - p[0:1, :]
    dy = q[:, 1:2] - p[1:2, :]
    dz = q[:, 2:3] - p[2:3, :]
    d_w = (dx * dx + dy * dy) + dz * dz

    work = d
    r = jnp.zeros_like(d)
    norm = jnp.zeros((q.shape[0], 1), jnp.float32)
    for _ in range(3):
        mk = jnp.min(work, axis=1, keepdims=True)
        m_eq = work == mk
        ek = jnp.min(jnp.where(m_eq, d_w, jnp.inf), axis=1, keepdims=True)
        rk = jax.lax.rsqrt(jnp.maximum(ek, 1e-20))
        r = jnp.where(m_eq, rk, r)
        norm = norm + rk
        work = jnp.where(m_eq, jnp.inf, work)
    flow2 = (
        jax.lax.dot_general(
            r, f, (((1,), (0,)), ((), ())),
            preferred_element_type=jnp.float32,
        )
        / norm
    )
    o_ref[0] = q - flow2


@jax.jit
def kernel(xyz1, xyz2, flow1):
    B, C, N1 = xyz1.shape
    N2 = xyz2.shape[2]
    mq = 512

    p_nat = xyz1 + flow1
    q_t = xyz2.transpose(0, 2, 1)
    f_t = flow1.transpose(0, 2, 1)

    out = pl.pallas_call(
        functools.partial(_warp_block, n1=N1),
        grid=(B, N2 // mq),
        in_specs=[
            pl.BlockSpec((1, mq, C), lambda b, i: (b, i, 0)),
            pl.BlockSpec((1, C, N1), lambda b, i: (b, 0, 0)),
            pl.BlockSpec((1, N1, C), lambda b, i: (b, 0, 0)),
        ],
        out_specs=pl.BlockSpec((1, mq, C), lambda b, i: (b, i, 0)),
        out_shape=jax.ShapeDtypeStruct((B, N2, C), jnp.float32),
        compiler_params=pltpu.CompilerParams(
            dimension_semantics=("parallel", "parallel")
        ),
    )(q_t, p_nat, f_t)
    return out.transpose(0, 2, 1)


# speedup vs baseline: 3.2151x; 1.1642x over previous
import functools

import jax
import jax.numpy as jnp
from jax.experimental import pallas as pl
from jax.experimental.pallas import tpu as pltpu


def _warp_block(q_ref, p_ref, f_ref, o_ref, *, n1):
    q = q_ref[0]
    p = p_ref[0]
    f = f_ref[0]

    q_hi = q.astype(jnp.bfloat16)
    p_hi = p.astype(jnp.bfloat16)
    qp = jax.lax.dot_general(
        q_hi, p_hi,
        (((1,), (0,)), ((), ())),
        preferred_element_type=jnp.float32,
    )
    q_lo = (q - q_hi.astype(jnp.float32)).astype(jnp.bfloat16)
    p_lo = (p - p_hi.astype(jnp.float32)).astype(jnp.bfloat16)
    qp_hi = jax.lax.dot_general(
        jnp.concatenate([q_hi, q_lo], axis=1),
        jnp.concatenate([p_hi, p_lo], axis=0),
        (((1,), (0,)), ((), ())),
        preferred_element_type=jnp.float32,
    )
    p2 = jnp.sum(p * p, axis=0, keepdims=True)
    q2 = jnp.sum(q * q, axis=1, keepdims=True)
    s = q2 + p2
    d = s - 2.0 * qp
    d_w = s - 2.0 * qp_hi

    work = d
    r = jnp.zeros_like(d)
    norm = jnp.zeros((q.shape[0], 1), jnp.float32)
    for _ in range(3):
        mk = jnp.min(work, axis=1, keepdims=True)
        m_eq = work == mk
        ek = jnp.min(jnp.where(m_eq, d_w, jnp.inf), axis=1, keepdims=True)
        rk = jax.lax.rsqrt(jnp.maximum(ek, 1e-20))
        r = jnp.where(m_eq, rk, r)
        norm = norm + rk
        work = jnp.where(m_eq, jnp.inf, work)
    flow2 = (
        jax.lax.dot_general(
            r, f, (((1,), (0,)), ((), ())),
            preferred_element_type=jnp.float32,
        )
        / norm
    )
    o_ref[0] = q - flow2


@jax.jit
def kernel(xyz1, xyz2, flow1):
    B, C, N1 = xyz1.shape
    N2 = xyz2.shape[2]
    mq = 512

    p_nat = xyz1 + flow1
    q_t = xyz2.transpose(0, 2, 1)
    f_t = flow1.transpose(0, 2, 1)

    out = pl.pallas_call(
        functools.partial(_warp_block, n1=N1),
        grid=(B, N2 // mq),
        in_specs=[
            pl.BlockSpec((1, mq, C), lambda b, i: (b, i, 0)),
            pl.BlockSpec((1, C, N1), lambda b, i: (b, 0, 0)),
            pl.BlockSpec((1, N1, C), lambda b, i: (b, 0, 0)),
        ],
        out_specs=pl.BlockSpec((1, mq, C), lambda b, i: (b, i, 0)),
        out_shape=jax.ShapeDtypeStruct((B, N2, C), jnp.float32),
        compiler_params=pltpu.CompilerParams(
            dimension_semantics=("parallel", "parallel")
        ),
    )(q_t, p_nat, f_t)
    return out.transpose(0, 2, 1)
